# baseline (speedup 1.0000x reference)
import functools

import jax
import jax.numpy as jnp
from jax import lax
from jax.experimental import pallas as pl
from jax.experimental.pallas import tpu as pltpu
from jax.experimental.pallas import tpu_sc as plsc

_SU, _SL, _TU, _TL = 1000.0, 0.0, 500.0, 0.0
_TM = 50
_DPAD = 128


def _sc_gather(table, idx, n_rows):
    info = plsc.get_sparse_core_info()
    nw = info.num_cores * info.num_subcores
    b_per_w = n_rows // nw
    d = table.shape[1]
    mesh = plsc.VectorSubcoreMesh(core_axis_name="c", subcore_axis_name="s")

    @functools.partial(
        pl.kernel, mesh=mesh,
        out_type=jax.ShapeDtypeStruct((n_rows, d), jnp.float32),
        scratch_types=[
            pltpu.VMEM((b_per_w,), jnp.int32),
            pltpu.VMEM((b_per_w, d), jnp.float32),
            pltpu.SemaphoreType.DMA,
        ],
    )
    def k(table_hbm, idx_hbm, out_hbm, idx_v, rows_v, sem):
        wid = lax.axis_index("s") * info.num_cores + lax.axis_index("c")
        base = wid * b_per_w
        pltpu.sync_copy(idx_hbm.at[pl.ds(base, b_per_w)], idx_v)
        pltpu.async_copy(table_hbm.at[idx_v], rows_v, sem).wait()
        pltpu.sync_copy(rows_v, out_hbm.at[pl.ds(base, b_per_w)])

    return k(table, idx)


def _expand_kernel(len_ref, lmax_ref,
                   ds_ref, su_ref, sl_ref, tu_ref, tl_ref,
                   out_ref):
    b = pl.program_id(0)
    j = pl.program_id(1)
    _, tm, l, _ = out_ref.shape
    m0 = j * tm

    tlen = len_ref[b]
    lmax = lmax_ref[0]
    v2 = (jax.lax.broadcasted_iota(jnp.int32, (tm, 1), 0) + m0) < tlen
    col_ok = jax.lax.broadcasted_iota(jnp.int32, (tm, l), 1) < lmax
    ds = jnp.where(v2 & col_ok, ds_ref[0, :, :l], 0.0)

    esl = jnp.where(v2, sl_ref[1:2, :], sl_ref[0:1, :])
    esu = jnp.where(v2, su_ref[1:2, :], su_ref[0:1, :])
    etl = jnp.where(v2, tl_ref[1:2, :], tl_ref[0:1, :])
    etu = jnp.where(v2, tu_ref[1:2, :], tu_ref[0:1, :])
    a_v = (esl * _SU - esu * _SL) * (1.0 / (_SU - _SL)) + \
          (etl * _TU - etu * _TL) * (1.0 / (_TU - _TL))
    b_v = (esu - esl) * (1.0 / (_SU - _SL)) + \
          (etu - etl) * (1.0 / (_TU - _TL))

    out_ref[0] = a_v[:, None, :] + b_v[:, None, :] * ds[:, :, None]


def kernel(traj_loc, mat2, vec, traj_len, l_max, emb_su, emb_sl, emb_tu, emb_tl):
    del vec
    b_sz, m_sz = traj_loc.shape
    n_loc, l_sz = mat2.shape
    e_sz = emb_su.shape[1]
    tm = _TM if m_sz % _TM == 0 else m_sz
    steps_per_b = m_sz // tm
    grid = (b_sz, steps_per_b)

    table = jnp.pad(mat2, ((1, 0), (0, _DPAD - l_sz)))
    n_pairs = b_sz * m_sz
    nw = 32
    n_rows = ((n_pairs + 8 * nw - 1) // (8 * nw)) * (8 * nw)
    idx = jnp.pad(traj_loc.astype(jnp.int32).reshape(-1),
                  ((0, n_rows - n_pairs),))
    ds_rows = _sc_gather(table, idx, n_rows)
    ds3 = ds_rows[:n_pairs].reshape(n_pairs // tm, tm, _DPAD)

    lmax_arr = jnp.asarray(l_max, jnp.int32).reshape(1)
    full = lambda bb, jj, *refs: (0, 0)

    out = pl.pallas_call(
        _expand_kernel,
        grid_spec=pltpu.PrefetchScalarGridSpec(
            num_scalar_prefetch=2,
            grid=grid,
            in_specs=[
                pl.BlockSpec((1, tm, _DPAD),
                             lambda bb, jj, *refs: (bb * steps_per_b + jj, 0, 0)),
                pl.BlockSpec((2, e_sz), full),
                pl.BlockSpec((2, e_sz), full),
                pl.BlockSpec((2, e_sz), full),
                pl.BlockSpec((2, e_sz), full),
            ],
            out_specs=pl.BlockSpec((1, tm, l_sz, e_sz),
                                   lambda bb, jj, *refs: (bb, jj, 0, 0)),
        ),
        out_shape=jax.ShapeDtypeStruct((b_sz, m_sz, l_sz, e_sz), jnp.float32),
    )(traj_len.astype(jnp.int32), lmax_arr,
      ds3, emb_su, emb_sl, emb_tu, emb_tl)
    return out
